# Initial kernel scaffold; baseline (speedup 1.0000x reference)
import jax
import jax.numpy as jnp
from jax.experimental import pallas as pl


def kernel(x, edge_index, Wl1, Wr1, b1, Wl2, Wr2, b2):
    raise NotImplementedError("write your pallas kernel here")


# speedup vs baseline: 8.0087x; 8.0087x over previous
import functools

import jax
import jax.numpy as jnp
from jax import lax
from jax.experimental import pallas as pl
from jax.experimental.pallas import tpu as pltpu
from jax.experimental.pallas import tpu_sc as plsc

_N = 10000
_E = 320000
_D_IN = 128
_D_H = 128
_D_OUT = 64

_NC = 2
_NS = 16
_NW = _NC * _NS
_CHUNK = 128
_CPW = 80
_E_PAD = _NW * _CPW * _CHUNK
_CSTG = 16
_RPT = 632
_N_PAD = _NS * _RPT
_CNT_N = 10240
_CSLC = _CNT_N // _NS


def _make_sc_segsum(d, with_counts):
  mesh = plsc.VectorSubcoreMesh(
      core_axis_name="c", subcore_axis_name="s",
      num_cores=_NC, num_subcores=_NS)

  out_type = [jax.ShapeDtypeStruct((_NC * _N_PAD, d), jnp.float32)]
  scratch = [
      pltpu.VMEM((_CSTG, _CHUNK), jnp.int32),
      pltpu.VMEM((_CSTG, _CHUNK), jnp.int32),
      pltpu.VMEM((_CHUNK, d), jnp.float32),
      pltpu.VMEM_SHARED((_N_PAD, d), jnp.float32),
      pltpu.SemaphoreType.DMA,
  ]
  if with_counts:
    out_type.append(jax.ShapeDtypeStruct((_NC, _CNT_N), jnp.float32))
    scratch += [
        pltpu.VMEM((_CNT_N,), jnp.float32),
        pltpu.VMEM((_CSLC,), jnp.float32),
        pltpu.VMEM((_CSLC,), jnp.float32),
        pltpu.VMEM_SHARED((_NS, _CNT_N), jnp.float32),
    ]

  def body(*refs):
    if with_counts:
      (table, srcs, dsts, zeros,
       out_s, out_c, src_v, dst_v, rows_v, acc_sh, sem,
       cnt_v, peer_v, comb_v, stage_sh) = refs
    else:
      (table, srcs, dsts, zeros,
       out_s, src_v, dst_v, rows_v, acc_sh, sem) = refs

    c = lax.axis_index("c")
    s = lax.axis_index("s")
    wid = c * _NS + s
    r0 = s * _RPT

    def copy_rows(src, dst, base_src, base_dst, buf):
      for k, sz in ((0, 128), (128, 128), (256, 128), (384, 128), (512, 120)):
        pltpu.sync_copy(src.at[pl.ds(base_src + k, sz)], buf.at[pl.ds(0, sz)])
        pltpu.sync_copy(buf.at[pl.ds(0, sz)], dst.at[pl.ds(base_dst + k, sz)])

    copy_rows(zeros, acc_sh, r0, r0, rows_v)
    if with_counts:
      def zero_hist(i, carry):
        cnt_v[pl.ds(i * 16, 16)] = jnp.zeros((16,), jnp.float32)
        return carry
      lax.fori_loop(0, _CNT_N // 16, zero_hist, 0)
    plsc.subcore_barrier()

    def outer(ph, carry):
      i0 = wid * _CPW + ph * _CSTG
      pltpu.sync_copy(srcs.at[pl.ds(i0, _CSTG)], src_v)
      pltpu.sync_copy(dsts.at[pl.ds(i0, _CSTG)], dst_v)

      def step(j, carry2):
        pltpu.async_copy(table.at[src_v.at[j]], rows_v, sem).wait()
        pltpu.sync_copy(rows_v, acc_sh.at[dst_v.at[j]], add=True)
        if with_counts:
          for k in range(_CHUNK // 16):
            dd = dst_v[j, pl.ds(k * 16, 16)]
            cnt, last = plsc.scan_count(dd)
            plsc.addupdate_scatter(cnt_v, [dd], cnt.astype(jnp.float32),
                                   mask=last)
        return carry2

      lax.fori_loop(0, _CSTG, step, 0)
      return carry

    lax.fori_loop(0, _CPW // _CSTG, outer, 0)
    plsc.subcore_barrier()

    if with_counts:
      pltpu.sync_copy(cnt_v, stage_sh.at[s])
      plsc.subcore_barrier()
      b0 = s * _CSLC

      def zero_comb(i, carry):
        comb_v[pl.ds(i * 16, 16)] = jnp.zeros((16,), jnp.float32)
        return carry
      lax.fori_loop(0, _CSLC // 16, zero_comb, 0)
      for t in range(_NS):
        pltpu.sync_copy(stage_sh.at[t, pl.ds(b0, _CSLC)], peer_v)

        def addup(i, carry):
          comb_v[pl.ds(i * 16, 16)] = (
              comb_v[pl.ds(i * 16, 16)] + peer_v[pl.ds(i * 16, 16)])
          return carry
        lax.fori_loop(0, _CSLC // 16, addup, 0)
      pltpu.sync_copy(comb_v, out_c.at[c, pl.ds(b0, _CSLC)])

    o0 = c * _N_PAD + r0
    copy_rows(acc_sh, out_s, r0, o0, rows_v)

  return pl.kernel(
      body, out_type=out_type, mesh=mesh, scratch_types=scratch,
      compiler_params=pltpu.CompilerParams(needs_layout_passes=False))


_BLK = 2048
_GRID = (_N + _BLK - 1) // _BLK


def _tc_dense1(x, s1a, s1b, ca, cb, Wl1, Wr1, b1, Wr2, b2):
  def tc_body(x_r, sa_r, sb_r, ca_r, cb_r, wl1_r, wr1_r, b1_r,
              wr2_r, b2_r, h_r, r2_r):
    cnt = jnp.maximum(ca_r[:, 0:1] + cb_r[:, 0:1], 1.0)
    mean = (sa_r[...] + sb_r[...]) / cnt
    dn = (((1,), (1,)), ((), ()))
    h = lax.dot_general(mean, wl1_r[...], dn,
                        preferred_element_type=jnp.float32)
    h = h + b1_r[...] + lax.dot_general(x_r[...], wr1_r[...], dn,
                                        preferred_element_type=jnp.float32)
    h = jnp.maximum(h, 0.0)
    h_r[...] = h
    r2_r[...] = lax.dot_general(h, wr2_r[...], dn,
                                preferred_element_type=jnp.float32) + b2_r[...]

  full = lambda shape: pl.BlockSpec(shape, lambda i: (0, 0))
  row = lambda width: pl.BlockSpec((_BLK, width), lambda i: (i, 0))
  return pl.pallas_call(
      tc_body,
      grid=(_GRID,),
      in_specs=[row(_D_IN), row(_D_H), row(_D_H), row(16), row(16),
                full((_D_H, _D_IN)), full((_D_H, _D_IN)), full((1, _D_H)),
                full((_D_OUT, _D_H)), full((1, _D_OUT))],
      out_specs=[row(_D_H), row(_D_OUT)],
      out_shape=[jax.ShapeDtypeStruct((_N, _D_H), jnp.float32),
                 jax.ShapeDtypeStruct((_N, _D_OUT), jnp.float32)],
  )(x, s1a, s1b, ca, cb, Wl1, Wr1, b1, Wr2, b2)


def _tc_dense2(s2a, s2b, ca, cb, r2, Wl2):
  def tc_body(sa_r, sb_r, ca_r, cb_r, r2_r, wl2_r, o_r):
    cnt = jnp.maximum(ca_r[:, 0:1] + cb_r[:, 0:1], 1.0)
    mean = (sa_r[...] + sb_r[...]) / cnt
    dn = (((1,), (1,)), ((), ()))
    o = lax.dot_general(mean, wl2_r[...], dn,
                        preferred_element_type=jnp.float32) + r2_r[...]
    o = o - jnp.max(o, axis=1, keepdims=True)
    lse = jnp.log(jnp.sum(jnp.exp(o), axis=1, keepdims=True))
    o_r[...] = o - lse

  full = lambda shape: pl.BlockSpec(shape, lambda i: (0, 0))
  row = lambda width: pl.BlockSpec((_BLK, width), lambda i: (i, 0))
  return pl.pallas_call(
      tc_body,
      grid=(_GRID,),
      in_specs=[row(_D_H), row(_D_H), row(16), row(16), row(_D_OUT),
                full((_D_OUT, _D_H))],
      out_specs=row(_D_OUT),
      out_shape=jax.ShapeDtypeStruct((_N, _D_OUT), jnp.float32),
  )(s2a, s2b, ca, cb, r2, Wl2)


@jax.jit
def kernel(x, edge_index, Wl1, Wr1, b1, Wl2, Wr2, b2):
  src = edge_index[0]
  dst = edge_index[1]
  npad = _E_PAD - _E
  pad_i = jnp.arange(npad, dtype=jnp.int32)
  srcs = jnp.concatenate([src, pad_i % _CHUNK]).reshape(_NW * _CPW, _CHUNK)
  dsts = jnp.concatenate([dst, _N + pad_i % (_N_PAD - _N)]
                         ).reshape(_NW * _CPW, _CHUNK)

  zeros_h = jnp.zeros((_N_PAD, _D_H), jnp.float32)

  sc1 = _make_sc_segsum(_D_H, with_counts=True)
  sums1, cnts = sc1(x, srcs, dsts, zeros_h)

  def cexp(part):
    return jnp.broadcast_to(part[:_N, None], (_N, 16))

  ca, cb = cexp(cnts[0]), cexp(cnts[1])
  s1a, s1b = sums1[:_N], sums1[_N_PAD:_N_PAD + _N]

  h, r2 = _tc_dense1(x, s1a, s1b, ca, cb,
                     Wl1, Wr1, b1.reshape(1, _D_H),
                     Wr2, b2.reshape(1, _D_OUT))

  sc2 = _make_sc_segsum(_D_H, with_counts=False)
  (sums2,) = sc2(h, srcs, dsts, zeros_h)
  s2a, s2b = sums2[:_N], sums2[_N_PAD:_N_PAD + _N]

  return _tc_dense2(s2a, s2b, ca, cb, r2, Wl2)


# speedup vs baseline: 11.3365x; 1.4155x over previous
import functools

import jax
import jax.numpy as jnp
from jax import lax
from jax.experimental import pallas as pl
from jax.experimental.pallas import tpu as pltpu
from jax.experimental.pallas import tpu_sc as plsc

_N = 10000
_E = 320000
_D_IN = 128
_D_H = 128
_D_OUT = 64

_NC = 2
_NS = 16
_NW = _NC * _NS
_CHUNK = 128
_CPW = 80
_E_PAD = _NW * _CPW * _CHUNK
_CSTG = 16
_RPT = 632
_N_PAD = _NS * _RPT
_CNT_N = 10240
_CSLC = _CNT_N // _NS


def _make_sc_segsum(d, with_counts):
  mesh = plsc.VectorSubcoreMesh(
      core_axis_name="c", subcore_axis_name="s",
      num_cores=_NC, num_subcores=_NS)

  out_type = [jax.ShapeDtypeStruct((_NC * _N_PAD, d), jnp.float32)]
  scratch = [
      pltpu.VMEM((_CSTG, _CHUNK), jnp.int32),
      pltpu.VMEM((_CSTG, _CHUNK), jnp.int32),
      pltpu.VMEM((_CHUNK, d), jnp.float32),
      pltpu.VMEM((_CHUNK, d), jnp.float32),
      pltpu.VMEM_SHARED((_N_PAD, d), jnp.float32),
      pltpu.SemaphoreType.DMA,
      pltpu.SemaphoreType.DMA,
  ]
  if with_counts:
    out_type.append(jax.ShapeDtypeStruct((_NC, _CNT_N), jnp.float32))
    out_type.append(jax.ShapeDtypeStruct((_NW, _CNT_N), jnp.float32))
    scratch += [
        pltpu.VMEM((_CNT_N,), jnp.float32),
        pltpu.VMEM((_CSLC,), jnp.float32),
        pltpu.VMEM((_CSLC,), jnp.float32),
    ]

  def body(*refs):
    if with_counts:
      (table, srcs, dsts,
       out_s, out_c, out_hist, src_v, dst_v, rows_a, rows_b, acc_sh,
       sem_a, sem_b, cnt_v, peer_v, comb_v) = refs
    else:
      (table, srcs, dsts,
       out_s, src_v, dst_v, rows_a, rows_b, acc_sh, sem_a, sem_b) = refs

    c = lax.axis_index("c")
    s = lax.axis_index("s")
    wid = c * _NS + s
    r0 = s * _RPT

    def zrow(i, carry):
      for k in range(d // 16):
        rows_a[i, pl.ds(k * 16, 16)] = jnp.zeros((16,), jnp.float32)
      return carry
    lax.fori_loop(0, _CHUNK, zrow, 0)
    for k, sz in ((0, 128), (128, 128), (256, 128), (384, 128), (512, 120)):
      pltpu.sync_copy(rows_a.at[pl.ds(0, sz)], acc_sh.at[pl.ds(r0 + k, sz)])
    if with_counts:
      def zero_hist(i, carry):
        cnt_v[pl.ds(i * 16, 16)] = jnp.zeros((16,), jnp.float32)
        return carry
      lax.fori_loop(0, _CNT_N // 16, zero_hist, 0)
    plsc.subcore_barrier()

    bufs = (rows_a, rows_b)
    sems = (sem_a, sem_b)

    def block(ph, carry):
      i0 = wid * _CPW + ph * _CSTG
      pltpu.sync_copy(srcs.at[pl.ds(i0, _CSTG)], src_v)
      pltpu.sync_copy(dsts.at[pl.ds(i0, _CSTG)], dst_v)
      descs = [None, None]

      def start(j):
        descs[j & 1] = pltpu.async_copy(
            table.at[src_v.at[j]], bufs[j & 1], sems[j & 1])

      def finish(j):
        descs[j & 1].wait()
        pltpu.sync_copy(bufs[j & 1], acc_sh.at[dst_v.at[j]], add=True)
        if with_counts:
          for k in range(_CHUNK // 16):
            dd = dst_v[j, pl.ds(k * 16, 16)]
            cnt, last = plsc.scan_count(dd)
            plsc.addupdate_scatter(cnt_v, [dd], cnt.astype(jnp.float32),
                                   mask=last)

      start(0)
      for j in range(_CSTG):
        if j + 1 < _CSTG:
          start(j + 1)
        finish(j)
      return carry

    lax.fori_loop(0, _CPW // _CSTG, block, 0)
    plsc.subcore_barrier()

    if with_counts:
      pltpu.sync_copy(cnt_v, out_hist.at[wid])
      plsc.subcore_barrier()
      b0 = s * _CSLC

      def zero_comb(i, carry):
        comb_v[pl.ds(i * 16, 16)] = jnp.zeros((16,), jnp.float32)
        return carry
      lax.fori_loop(0, _CSLC // 16, zero_comb, 0)
      for t in range(_NS):
        pltpu.sync_copy(out_hist.at[c * _NS + t, pl.ds(b0, _CSLC)], peer_v)

        def addup(i, carry):
          comb_v[pl.ds(i * 16, 16)] = (
              comb_v[pl.ds(i * 16, 16)] + peer_v[pl.ds(i * 16, 16)])
          return carry
        lax.fori_loop(0, _CSLC // 16, addup, 0)
      pltpu.sync_copy(comb_v, out_c.at[c, pl.ds(b0, _CSLC)])

    o0 = c * _N_PAD + r0
    for k, sz in ((0, 128), (128, 128), (256, 128), (384, 128), (512, 120)):
      pltpu.sync_copy(acc_sh.at[pl.ds(r0 + k, sz)], rows_a.at[pl.ds(0, sz)])
      pltpu.sync_copy(rows_a.at[pl.ds(0, sz)], out_s.at[pl.ds(o0 + k, sz)])

  return pl.kernel(
      body, out_type=out_type, mesh=mesh, scratch_types=scratch,
      compiler_params=pltpu.CompilerParams(needs_layout_passes=False))


_BLK = 2048
_GRID = (_N + _BLK - 1) // _BLK


def _tc_dense1(x, s1a, s1b, ca, cb, Wl1, Wr1, b1, Wr2, b2):
  def tc_body(x_r, sa_r, sb_r, ca_r, cb_r, wl1_r, wr1_r, b1_r,
              wr2_r, b2_r, h_r, r2_r):
    cnt = jnp.maximum(ca_r[:, 0:1] + cb_r[:, 0:1], 1.0)
    mean = (sa_r[...] + sb_r[...]) / cnt
    dn = (((1,), (1,)), ((), ()))
    h = lax.dot_general(mean, wl1_r[...], dn,
                        preferred_element_type=jnp.float32)
    h = h + b1_r[...] + lax.dot_general(x_r[...], wr1_r[...], dn,
                                        preferred_element_type=jnp.float32)
    h = jnp.maximum(h, 0.0)
    h_r[...] = h
    r2_r[...] = lax.dot_general(h, wr2_r[...], dn,
                                preferred_element_type=jnp.float32) + b2_r[...]

  full = lambda shape: pl.BlockSpec(shape, lambda i: (0, 0))
  row = lambda width: pl.BlockSpec((_BLK, width), lambda i: (i, 0))
  return pl.pallas_call(
      tc_body,
      grid=(_GRID,),
      in_specs=[row(_D_IN), row(_D_H), row(_D_H), row(16), row(16),
                full((_D_H, _D_IN)), full((_D_H, _D_IN)), full((1, _D_H)),
                full((_D_OUT, _D_H)), full((1, _D_OUT))],
      out_specs=[row(_D_H), row(_D_OUT)],
      out_shape=[jax.ShapeDtypeStruct((_N, _D_H), jnp.float32),
                 jax.ShapeDtypeStruct((_N, _D_OUT), jnp.float32)],
  )(x, s1a, s1b, ca, cb, Wl1, Wr1, b1, Wr2, b2)


def _tc_dense2(s2a, s2b, ca, cb, r2, Wl2):
  def tc_body(sa_r, sb_r, ca_r, cb_r, r2_r, wl2_r, o_r):
    cnt = jnp.maximum(ca_r[:, 0:1] + cb_r[:, 0:1], 1.0)
    mean = (sa_r[...] + sb_r[...]) / cnt
    dn = (((1,), (1,)), ((), ()))
    o = lax.dot_general(mean, wl2_r[...], dn,
                        preferred_element_type=jnp.float32) + r2_r[...]
    o = o - jnp.max(o, axis=1, keepdims=True)
    lse = jnp.log(jnp.sum(jnp.exp(o), axis=1, keepdims=True))
    o_r[...] = o - lse

  full = lambda shape: pl.BlockSpec(shape, lambda i: (0, 0))
  row = lambda width: pl.BlockSpec((_BLK, width), lambda i: (i, 0))
  return pl.pallas_call(
      tc_body,
      grid=(_GRID,),
      in_specs=[row(_D_H), row(_D_H), row(16), row(16), row(_D_OUT),
                full((_D_OUT, _D_H))],
      out_specs=row(_D_OUT),
      out_shape=jax.ShapeDtypeStruct((_N, _D_OUT), jnp.float32),
  )(s2a, s2b, ca, cb, r2, Wl2)


@jax.jit
def kernel(x, edge_index, Wl1, Wr1, b1, Wl2, Wr2, b2):
  src = edge_index[0]
  dst = edge_index[1]
  npad = _E_PAD - _E
  pad_i = jnp.arange(npad, dtype=jnp.int32)
  srcs = jnp.concatenate([src, pad_i % _CHUNK]).reshape(_NW * _CPW, _CHUNK)
  dsts = jnp.concatenate([dst, _N + pad_i % (_N_PAD - _N)]
                         ).reshape(_NW * _CPW, _CHUNK)

  sc1 = _make_sc_segsum(_D_H, with_counts=True)
  sums1, cnts, _hist = sc1(x, srcs, dsts)

  def cexp(part):
    return jnp.broadcast_to(part[:_N, None], (_N, 16))

  ca, cb = cexp(cnts[0]), cexp(cnts[1])
  s1a, s1b = sums1[:_N], sums1[_N_PAD:_N_PAD + _N]

  h, r2 = _tc_dense1(x, s1a, s1b, ca, cb,
                     Wl1, Wr1, b1.reshape(1, _D_H),
                     Wr2, b2.reshape(1, _D_OUT))

  sc2 = _make_sc_segsum(_D_H, with_counts=False)
  (sums2,) = sc2(h, srcs, dsts)
  s2a, s2b = sums2[:_N], sums2[_N_PAD:_N_PAD + _N]

  return _tc_dense2(s2a, s2b, ca, cb, r2, Wl2)


# speedup vs baseline: 12.3403x; 1.0885x over previous
import functools

import jax
import jax.numpy as jnp
from jax import lax
from jax.experimental import pallas as pl
from jax.experimental.pallas import tpu as pltpu
from jax.experimental.pallas import tpu_sc as plsc

_N = 10000
_E = 320000
_D_IN = 128
_D_H = 128
_D_OUT = 64

_NC = 2
_NS = 16
_NW = _NC * _NS
_CHUNK = 128
_CPW = 80
_E_PAD = _NW * _CPW * _CHUNK
_CSTG = 16
_RPT = 632
_N_PAD = _NS * _RPT
_CNT_N = 10240
_CSLC = _CNT_N // _NS


def _make_sc_segsum(d, with_counts):
  mesh = plsc.VectorSubcoreMesh(
      core_axis_name="c", subcore_axis_name="s",
      num_cores=_NC, num_subcores=_NS)

  out_type = [jax.ShapeDtypeStruct((_NC * _N_PAD, d), jnp.float32)]
  scratch = [
      pltpu.VMEM((_CSTG, _CHUNK), jnp.int32),
      pltpu.VMEM((_CSTG, _CHUNK), jnp.int32),
      pltpu.VMEM((_CHUNK, d), jnp.float32),
      pltpu.VMEM((_CHUNK, d), jnp.float32),
      pltpu.VMEM_SHARED((_N_PAD, d), jnp.float32),
      pltpu.SemaphoreType.DMA,
      pltpu.SemaphoreType.DMA,
  ]
  if with_counts:
    out_type.append(jax.ShapeDtypeStruct((_NC, _CNT_N), jnp.float32))
    out_type.append(jax.ShapeDtypeStruct((_NW, _CNT_N), jnp.float32))
    scratch += [
        pltpu.VMEM((_CNT_N,), jnp.float32),
        pltpu.VMEM((_CSLC,), jnp.float32),
        pltpu.VMEM((_CSLC,), jnp.float32),
    ]

  def body(*refs):
    if with_counts:
      (table, srcs, dsts,
       out_s, out_c, out_hist, src_v, dst_v, rows_a, rows_b, acc_sh,
       sem_a, sem_b, cnt_v, peer_v, comb_v) = refs
    else:
      (table, srcs, dsts,
       out_s, src_v, dst_v, rows_a, rows_b, acc_sh, sem_a, sem_b) = refs

    c = lax.axis_index("c")
    s = lax.axis_index("s")
    wid = c * _NS + s
    r0 = s * _RPT

    def zrow(i, carry):
      for k in range(d // 16):
        rows_a[i, pl.ds(k * 16, 16)] = jnp.zeros((16,), jnp.float32)
      return carry
    lax.fori_loop(0, _CHUNK, zrow, 0)
    for k, sz in ((0, 128), (128, 128), (256, 128), (384, 128), (512, 120)):
      pltpu.sync_copy(rows_a.at[pl.ds(0, sz)], acc_sh.at[pl.ds(r0 + k, sz)])
    if with_counts:
      def zero_hist(i, carry):
        cnt_v[pl.ds(i * 16, 16)] = jnp.zeros((16,), jnp.float32)
        return carry
      lax.fori_loop(0, _CNT_N // 16, zero_hist, 0)
    plsc.subcore_barrier()

    bufs = (rows_a, rows_b)
    sems = (sem_a, sem_b)

    def block(ph, carry):
      i0 = wid * _CPW + ph * _CSTG
      pltpu.sync_copy(srcs.at[pl.ds(i0, _CSTG)], src_v)
      pltpu.sync_copy(dsts.at[pl.ds(i0, _CSTG)], dst_v)
      descs = [None, None]

      def start(j):
        descs[j & 1] = pltpu.async_copy(
            table.at[src_v.at[j]], bufs[j & 1], sems[j & 1])

      def finish(j):
        descs[j & 1].wait()
        pltpu.sync_copy(bufs[j & 1], acc_sh.at[dst_v.at[j]], add=True)
        if with_counts:
          for k in range(_CHUNK // 16):
            dd = dst_v[j, pl.ds(k * 16, 16)]
            cnt, last = plsc.scan_count(dd)
            plsc.addupdate_scatter(cnt_v, [dd], cnt.astype(jnp.float32),
                                   mask=last)

      start(0)
      for j in range(_CSTG):
        if j + 1 < _CSTG:
          start(j + 1)
        finish(j)
      return carry

    lax.fori_loop(0, _CPW // _CSTG, block, 0)
    plsc.subcore_barrier()

    if with_counts:
      pltpu.sync_copy(cnt_v, out_hist.at[wid])
      plsc.subcore_barrier()
      b0 = s * _CSLC

      def zero_comb(i, carry):
        comb_v[pl.ds(i * 16, 16)] = jnp.zeros((16,), jnp.float32)
        return carry
      lax.fori_loop(0, _CSLC // 16, zero_comb, 0)
      for t in range(_NS):
        pltpu.sync_copy(out_hist.at[c * _NS + t, pl.ds(b0, _CSLC)], peer_v)

        def addup(i, carry):
          comb_v[pl.ds(i * 16, 16)] = (
              comb_v[pl.ds(i * 16, 16)] + peer_v[pl.ds(i * 16, 16)])
          return carry
        lax.fori_loop(0, _CSLC // 16, addup, 0)
      pltpu.sync_copy(comb_v, out_c.at[c, pl.ds(b0, _CSLC)])

    o0 = c * _N_PAD + r0
    for k, sz in ((0, 128), (128, 128), (256, 128), (384, 128), (512, 120)):
      pltpu.sync_copy(acc_sh.at[pl.ds(r0 + k, sz)], rows_a.at[pl.ds(0, sz)])
      pltpu.sync_copy(rows_a.at[pl.ds(0, sz)], out_s.at[pl.ds(o0 + k, sz)])

  return pl.kernel(
      body, out_type=out_type, mesh=mesh, scratch_types=scratch,
      compiler_params=pltpu.CompilerParams(
          needs_layout_passes=False,
          use_tc_tiling_on_sc=(d % 128 == 0)))


_BLK = _RPT
_GRID = _NS
_full = lambda shape: pl.BlockSpec(shape, lambda i: (0, 0))
_row = lambda width: pl.BlockSpec((_BLK, width), lambda i: (i, 0))
_rowb = lambda width: pl.BlockSpec((_BLK, width), lambda i: (i + _NS, 0))


def _tc_dense1(x, sums1, ca, cb, Wl1, Wr1, b1, Wl2, Wr2, b2):
  def tc_body(x_r, sa_r, sb_r, ca_r, cb_r, wl1_r, wr1_r, b1_r,
              wl2_r, wr2_r, b2_r, p2_r, r2_r):
    cnt = jnp.maximum(ca_r[:, 0:1] + cb_r[:, 0:1], 1.0)
    mean = (sa_r[...] + sb_r[...]) / cnt
    dn = (((1,), (1,)), ((), ()))
    h = lax.dot_general(mean, wl1_r[...], dn,
                        preferred_element_type=jnp.float32)
    h = h + b1_r[...] + lax.dot_general(x_r[...], wr1_r[...], dn,
                                        preferred_element_type=jnp.float32)
    h = jnp.maximum(h, 0.0)
    p2_r[...] = lax.dot_general(h, wl2_r[...], dn,
                                preferred_element_type=jnp.float32)
    r2_r[...] = lax.dot_general(h, wr2_r[...], dn,
                                preferred_element_type=jnp.float32) + b2_r[...]

  return pl.pallas_call(
      tc_body,
      grid=(_GRID,),
      in_specs=[_row(_D_IN), _row(_D_H), _rowb(_D_H), _row(16), _row(16),
                _full((_D_H, _D_IN)), _full((_D_H, _D_IN)), _full((1, _D_H)),
                _full((_D_OUT, _D_H)), _full((_D_OUT, _D_H)),
                _full((1, _D_OUT))],
      out_specs=[_row(_D_OUT), _row(_D_OUT)],
      out_shape=[jax.ShapeDtypeStruct((_N, _D_OUT), jnp.float32),
                 jax.ShapeDtypeStruct((_N, _D_OUT), jnp.float32)],
  )(x, sums1, sums1, ca, cb, Wl1, Wr1, b1, Wl2, Wr2, b2)


def _tc_dense2(sums2, ca, cb, r2):
  def tc_body(sa_r, sb_r, ca_r, cb_r, r2_r, o_r):
    cnt = jnp.maximum(ca_r[:, 0:1] + cb_r[:, 0:1], 1.0)
    o = (sa_r[...] + sb_r[...]) / cnt + r2_r[...]
    o = o - jnp.max(o, axis=1, keepdims=True)
    lse = jnp.log(jnp.sum(jnp.exp(o), axis=1, keepdims=True))
    o_r[...] = o - lse

  return pl.pallas_call(
      tc_body,
      grid=(_GRID,),
      in_specs=[_row(_D_OUT), _rowb(_D_OUT), _row(16), _row(16),
                _row(_D_OUT)],
      out_specs=_row(_D_OUT),
      out_shape=jax.ShapeDtypeStruct((_N, _D_OUT), jnp.float32),
  )(sums2, sums2, ca, cb, r2)


@jax.jit
def kernel(x, edge_index, Wl1, Wr1, b1, Wl2, Wr2, b2):
  src = edge_index[0]
  dst = edge_index[1]
  npad = _E_PAD - _E
  pad_i = jnp.arange(npad, dtype=jnp.int32)
  srcs = jnp.concatenate([src, pad_i % _CHUNK]).reshape(_NW * _CPW, _CHUNK)
  dsts = jnp.concatenate([dst, _N + pad_i % (_N_PAD - _N)]
                         ).reshape(_NW * _CPW, _CHUNK)

  sc1 = _make_sc_segsum(_D_H, with_counts=True)
  sums1, cnts, _hist = sc1(x, srcs, dsts)

  def cexp(part):
    return jnp.broadcast_to(part[:_N, None], (_N, 16))

  ca, cb = cexp(cnts[0]), cexp(cnts[1])

  p2, r2 = _tc_dense1(x, sums1, ca, cb,
                      Wl1, Wr1, b1.reshape(1, _D_H),
                      Wl2, Wr2, b2.reshape(1, _D_OUT))

  sc2 = _make_sc_segsum(_D_OUT, with_counts=False)
  (sums2,) = sc2(p2, srcs, dsts)

  return _tc_dense2(sums2, ca, cb, r2)


# speedup vs baseline: 13.0423x; 1.0569x over previous
import functools

import jax
import jax.numpy as jnp
from jax import lax
from jax.experimental import pallas as pl
from jax.experimental.pallas import tpu as pltpu
from jax.experimental.pallas import tpu_sc as plsc

_N = 10000
_E = 320000
_D_IN = 128
_D_H = 128
_D_OUT = 64

_NC = 2
_NS = 16
_NW = _NC * _NS
_CHUNK = 128
_CPW = 80
_E_PAD = _NW * _CPW * _CHUNK
_CSTG = 16
_RPT = 632
_N_PAD = _NS * _RPT
_CNT_N = 10240
_CSLC = _CNT_N // _NS


def _make_sc_segsum(d, with_counts):
  mesh = plsc.VectorSubcoreMesh(
      core_axis_name="c", subcore_axis_name="s",
      num_cores=_NC, num_subcores=_NS)

  nbuf = 2 if d >= 128 else 4
  out_type = [jax.ShapeDtypeStruct((_NC * _N_PAD, d), jnp.float32)]
  scratch = (
      [pltpu.VMEM((_CSTG, _CHUNK), jnp.int32),
       pltpu.VMEM((_CSTG, _CHUNK), jnp.int32)]
      + [pltpu.VMEM((_CHUNK, d), jnp.float32)] * nbuf
      + [pltpu.VMEM_SHARED((_N_PAD, d), jnp.float32)]
      + [pltpu.SemaphoreType.DMA] * nbuf
  )
  if with_counts:
    out_type.append(jax.ShapeDtypeStruct((_NC, _CNT_N), jnp.float32))
    out_type.append(jax.ShapeDtypeStruct((_NW, _CNT_N), jnp.float32))
    scratch += [
        pltpu.VMEM((_CNT_N,), jnp.float32),
        pltpu.VMEM((_CSLC,), jnp.float32),
        pltpu.VMEM((_CSLC,), jnp.float32),
    ]

  def body(*refs):
    table, srcs, dsts = refs[0:3]
    if with_counts:
      out_s, out_c, out_hist = refs[3:6]
      p = 6
    else:
      out_s = refs[3]
      p = 4
    src_v, dst_v = refs[p:p + 2]
    p += 2
    bufs = refs[p:p + nbuf]
    p += nbuf
    acc_sh = refs[p]
    p += 1
    sems = refs[p:p + nbuf]
    p += nbuf
    if with_counts:
      cnt_v, peer_v, comb_v = refs[p:p + 3]
    rows_a = bufs[0]

    c = lax.axis_index("c")
    s = lax.axis_index("s")
    wid = c * _NS + s
    r0 = s * _RPT

    def zrow(i, carry):
      for k in range(d // 16):
        rows_a[i, pl.ds(k * 16, 16)] = jnp.zeros((16,), jnp.float32)
      return carry
    lax.fori_loop(0, _CHUNK, zrow, 0)
    for k, sz in ((0, 128), (128, 128), (256, 128), (384, 128), (512, 120)):
      pltpu.sync_copy(rows_a.at[pl.ds(0, sz)], acc_sh.at[pl.ds(r0 + k, sz)])
    if with_counts:
      def zero_hist(i, carry):
        cnt_v[pl.ds(i * 16, 16)] = jnp.zeros((16,), jnp.float32)
        return carry
      lax.fori_loop(0, _CNT_N // 16, zero_hist, 0)
    plsc.subcore_barrier()

    def block(ph, carry):
      i0 = wid * _CPW + ph * _CSTG
      pltpu.sync_copy(srcs.at[pl.ds(i0, _CSTG)], src_v)
      pltpu.sync_copy(dsts.at[pl.ds(i0, _CSTG)], dst_v)
      descs = [None] * nbuf

      nb = len(bufs)

      def start(j):
        descs[j % nb] = pltpu.async_copy(
            table.at[src_v.at[j]], bufs[j % nb], sems[j % nb])

      def finish(j):
        if with_counts:
          for k in range(_CHUNK // 16):
            dd = dst_v[j, pl.ds(k * 16, 16)]
            cnt, last = plsc.scan_count(dd)
            plsc.addupdate_scatter(cnt_v, [dd], cnt.astype(jnp.float32),
                                   mask=last)
        descs[j % nb].wait()
        pltpu.sync_copy(bufs[j % nb], acc_sh.at[dst_v.at[j]], add=True)

      for j in range(nb - 1):
        start(j)
      for j in range(_CSTG):
        if j + nb - 1 < _CSTG:
          start(j + nb - 1)
        finish(j)
      return carry

    lax.fori_loop(0, _CPW // _CSTG, block, 0)
    if with_counts:
      pltpu.async_copy(cnt_v, out_hist.at[wid], sems[0])
    plsc.subcore_barrier()

    o0 = c * _N_PAD + r0
    for k, sz in ((0, 128), (128, 128), (256, 128), (384, 128), (512, 120)):
      pltpu.sync_copy(acc_sh.at[pl.ds(r0 + k, sz)], rows_a.at[pl.ds(0, sz)])
      pltpu.sync_copy(rows_a.at[pl.ds(0, sz)], out_s.at[pl.ds(o0 + k, sz)])

    if with_counts:
      pltpu.make_async_copy(cnt_v, out_hist.at[wid], sems[0]).wait()
      plsc.subcore_barrier()
      b0 = s * _CSLC

      def zero_comb(i, carry):
        comb_v[pl.ds(i * 16, 16)] = jnp.zeros((16,), jnp.float32)
        return carry
      lax.fori_loop(0, _CSLC // 16, zero_comb, 0)
      for t in range(_NS):
        pltpu.sync_copy(out_hist.at[c * _NS + t, pl.ds(b0, _CSLC)], peer_v)

        def addup(i, carry):
          comb_v[pl.ds(i * 16, 16)] = (
              comb_v[pl.ds(i * 16, 16)] + peer_v[pl.ds(i * 16, 16)])
          return carry
        lax.fori_loop(0, _CSLC // 16, addup, 0)
      pltpu.sync_copy(comb_v, out_c.at[c, pl.ds(b0, _CSLC)])

  return pl.kernel(
      body, out_type=out_type, mesh=mesh, scratch_types=scratch,
      compiler_params=pltpu.CompilerParams(
          needs_layout_passes=False,
          use_tc_tiling_on_sc=(d % 128 == 0)))


_BLK = _RPT
_GRID = _NS
_full = lambda shape: pl.BlockSpec(shape, lambda i: (0, 0))
_row = lambda width: pl.BlockSpec((_BLK, width), lambda i: (i, 0))
_rowb = lambda width: pl.BlockSpec((_BLK, width), lambda i: (i + _NS, 0))


def _tc_dense1(x, sums1, ca, cb, Wl1, Wr1, b1, Wl2, Wr2, b2):
  def tc_body(x_r, sa_r, sb_r, ca_r, cb_r, wl1_r, wr1_r, b1_r,
              wl2_r, wr2_r, b2_r, p2_r, r2_r):
    cnt = jnp.maximum(ca_r[:, 0:1] + cb_r[:, 0:1], 1.0)
    mean = (sa_r[...] + sb_r[...]) / cnt
    dn = (((1,), (1,)), ((), ()))
    h = lax.dot_general(mean, wl1_r[...], dn,
                        preferred_element_type=jnp.float32)
    h = h + b1_r[...] + lax.dot_general(x_r[...], wr1_r[...], dn,
                                        preferred_element_type=jnp.float32)
    h = jnp.maximum(h, 0.0)
    p2_r[...] = lax.dot_general(h, wl2_r[...], dn,
                                preferred_element_type=jnp.float32)
    r2_r[...] = lax.dot_general(h, wr2_r[...], dn,
                                preferred_element_type=jnp.float32) + b2_r[...]

  return pl.pallas_call(
      tc_body,
      grid=(_GRID,),
      in_specs=[_row(_D_IN), _row(_D_H), _rowb(_D_H), _row(16), _row(16),
                _full((_D_H, _D_IN)), _full((_D_H, _D_IN)), _full((1, _D_H)),
                _full((_D_OUT, _D_H)), _full((_D_OUT, _D_H)),
                _full((1, _D_OUT))],
      out_specs=[_row(_D_OUT), _row(_D_OUT)],
      out_shape=[jax.ShapeDtypeStruct((_N, _D_OUT), jnp.float32),
                 jax.ShapeDtypeStruct((_N, _D_OUT), jnp.float32)],
  )(x, sums1, sums1, ca, cb, Wl1, Wr1, b1, Wl2, Wr2, b2)


def _tc_dense2(sums2, ca, cb, r2):
  def tc_body(sa_r, sb_r, ca_r, cb_r, r2_r, o_r):
    cnt = jnp.maximum(ca_r[:, 0:1] + cb_r[:, 0:1], 1.0)
    o = (sa_r[...] + sb_r[...]) / cnt + r2_r[...]
    o = o - jnp.max(o, axis=1, keepdims=True)
    lse = jnp.log(jnp.sum(jnp.exp(o), axis=1, keepdims=True))
    o_r[...] = o - lse

  return pl.pallas_call(
      tc_body,
      grid=(_GRID,),
      in_specs=[_row(_D_OUT), _rowb(_D_OUT), _row(16), _row(16),
                _row(_D_OUT)],
      out_specs=_row(_D_OUT),
      out_shape=jax.ShapeDtypeStruct((_N, _D_OUT), jnp.float32),
  )(sums2, sums2, ca, cb, r2)


@jax.jit
def kernel(x, edge_index, Wl1, Wr1, b1, Wl2, Wr2, b2):
  src = edge_index[0]
  dst = edge_index[1]
  npad = _E_PAD - _E
  pad_i = jnp.arange(npad, dtype=jnp.int32)
  srcs = jnp.concatenate([src, pad_i % _CHUNK]).reshape(_NW * _CPW, _CHUNK)
  dsts = jnp.concatenate([dst, _N + pad_i % (_N_PAD - _N)]
                         ).reshape(_NW * _CPW, _CHUNK)

  sc1 = _make_sc_segsum(_D_H, with_counts=True)
  sums1, cnts, _hist = sc1(x, srcs, dsts)

  def cexp(part):
    return jnp.broadcast_to(part[:_N, None], (_N, 16))

  ca, cb = cexp(cnts[0]), cexp(cnts[1])

  p2, r2 = _tc_dense1(x, sums1, ca, cb,
                      Wl1, Wr1, b1.reshape(1, _D_H),
                      Wl2, Wr2, b2.reshape(1, _D_OUT))

  sc2 = _make_sc_segsum(_D_OUT, with_counts=False)
  (sums2,) = sc2(p2, srcs, dsts)

  return _tc_dense2(sums2, ca, cb, r2)
